# baseline (speedup 1.0000x reference)
import functools
import math

import jax
import jax.numpy as jnp
from jax import lax
from jax.experimental import pallas as pl
from jax.experimental.pallas import tpu as pltpu
from jax.experimental.pallas import tpu_sc as plsc

NC = 2
NS = 16
NW = NC * NS
LANES = 16
EMB = 64
SCALE = math.sqrt(EMB)
ROW = 128
NBUF = 4


@jax.jit
def _lookup(tokens_t, table_pad):
    n_pos, n_rows = tokens_t.shape
    slab = n_rows // NW

    mesh = plsc.VectorSubcoreMesh(core_axis_name="c", subcore_axis_name="s")

    pair_bufs = [pltpu.VMEM((slab, ROW), jnp.float32) for _ in range(NBUF)]
    slab_bufs = [pltpu.VMEM((EMB, slab), jnp.float32) for _ in range(NBUF)]
    gsems = [pltpu.SemaphoreType.DMA for _ in range(NBUF)]
    ssems = [pltpu.SemaphoreType.DMA for _ in range(NBUF)]

    @functools.partial(
        pl.kernel,
        out_type=jax.ShapeDtypeStruct((n_pos, EMB, n_rows), jnp.float32),
        mesh=mesh,
        scratch_types=[pltpu.VMEM((n_pos, slab), jnp.int32)]
        + pair_bufs + slab_bufs + gsems + ssems,
        compiler_params=pltpu.CompilerParams(needs_layout_passes=False),
    )
    def body(tok_hbm, table_hbm, out_hbm, tok_v, *refs):
        rows = refs[:NBUF]
        slabs = refs[NBUF:2 * NBUF]
        gsem = refs[2 * NBUF:3 * NBUF]
        ssem = refs[3 * NBUF:4 * NBUF]

        wid = lax.axis_index("s") * NC + lax.axis_index("c")
        r0 = wid * slab

        pltpu.sync_copy(tok_hbm.at[:, pl.ds(r0, slab)], tok_v)

        def gather_desc(p, b):
            return pltpu.make_async_copy(
                table_hbm.at[tok_v.at[p]], rows[b], gsem[b]
            )

        def store_desc(p, b):
            dst = out_hbm.at[p, :, pl.ds(r0, slab)]
            return pltpu.make_async_copy(slabs[b], dst, ssem[b])

        def transpose_slab(b):
            @plsc.parallel_loop(0, slab // LANES, unroll=2)
            def _g(g):
                jvec = lax.iota(jnp.int32, LANES) + g * LANES
                for e in range(EMB):
                    col = jnp.full((LANES,), e, jnp.int32)
                    v = plsc.load_gather(rows[b], [jvec, col])
                    slabs[b][e, pl.ds(g * LANES, LANES)] = v

        for b in range(NBUF):
            gather_desc(b, b).start()

        @pl.loop(0, n_pos, step=NBUF)
        def _ring(p0):
            for b in range(NBUF):
                p = p0 + b
                gather_desc(p, b).wait()

                @pl.when(p >= NBUF)
                def _():
                    store_desc(p, b).wait()

                transpose_slab(b)

                @pl.when(p + NBUF < n_pos)
                def _():
                    gather_desc(p + NBUF, b).start()

                store_desc(p, b).start()

        for b in range(NBUF):
            store_desc(n_pos - NBUF + b, b).wait()

    return body(tokens_t, table_pad)


def kernel(tokens, table):
    if tokens.dtype != jnp.int32:
        tokens = tokens.astype(jnp.int32)
    n_vocab, emb = table.shape
    table_pad = jnp.pad(table * SCALE, ((0, 0), (0, ROW - emb)))
    out_t = _lookup(tokens.T, table_pad)
    return jnp.transpose(out_t, (2, 0, 1))


# speedup vs baseline: 1.1205x; 1.1205x over previous
import functools
import math

import jax
import jax.numpy as jnp
from jax import lax
from jax.experimental import pallas as pl
from jax.experimental.pallas import tpu as pltpu
from jax.experimental.pallas import tpu_sc as plsc

NC = 2
NS = 16
NW = NC * NS
LANES = 16
EMB = 64
SCALE = math.sqrt(EMB)
ROW = 128
NBUF = 4


@jax.jit
def _lookup(tokens_t, table_pad):
    n_pos, n_rows = tokens_t.shape
    slab = n_rows // NW

    mesh = plsc.VectorSubcoreMesh(core_axis_name="c", subcore_axis_name="s")

    pair_bufs = [pltpu.VMEM((slab, ROW), jnp.float32) for _ in range(NBUF)]
    slab_bufs = [pltpu.VMEM((EMB, slab), jnp.float32) for _ in range(NBUF)]
    gsems = [pltpu.SemaphoreType.DMA for _ in range(NBUF)]
    ssems = [pltpu.SemaphoreType.DMA for _ in range(NBUF)]

    @functools.partial(
        pl.kernel,
        out_type=jax.ShapeDtypeStruct((n_pos, EMB, n_rows), jnp.float32),
        mesh=mesh,
        scratch_types=[pltpu.VMEM((n_pos, slab), jnp.int32)]
        + pair_bufs + slab_bufs + gsems + ssems,
        compiler_params=pltpu.CompilerParams(needs_layout_passes=False),
    )
    def body(tok_hbm, table_hbm, out_hbm, tok_v, *refs):
        rows = refs[:NBUF]
        slabs = refs[NBUF:2 * NBUF]
        gsem = refs[2 * NBUF:3 * NBUF]
        ssem = refs[3 * NBUF:4 * NBUF]

        wid = lax.axis_index("s") * NC + lax.axis_index("c")
        r0 = wid * slab

        pltpu.sync_copy(tok_hbm.at[:, pl.ds(r0, slab)], tok_v)

        def gather_desc(p, b):
            return pltpu.make_async_copy(
                table_hbm.at[tok_v.at[p]], rows[b], gsem[b]
            )

        def store_desc(p, b):
            dst = out_hbm.at[p, :, pl.ds(r0, slab)]
            return pltpu.make_async_copy(slabs[b], dst, ssem[b])

        jvecs = [
            lax.iota(jnp.int32, LANES) + (g * LANES)
            for g in range(slab // LANES)
        ]

        def transpose_slab(b):
            @plsc.parallel_loop(0, EMB, unroll=8)
            def _e(e):
                col = jnp.full((LANES,), 0, jnp.int32) + e
                for g in range(slab // LANES):
                    v = plsc.load_gather(rows[b], [jvecs[g], col])
                    slabs[b][e, pl.ds(g * LANES, LANES)] = v

        for b in range(NBUF):
            gather_desc(b, b).start()

        @pl.loop(0, n_pos, step=NBUF)
        def _ring(p0):
            for b in range(NBUF):
                p = p0 + b
                gather_desc(p, b).wait()

                @pl.when(p >= NBUF)
                def _():
                    store_desc(p, b).wait()

                transpose_slab(b)

                @pl.when(p + NBUF < n_pos)
                def _():
                    gather_desc(p + NBUF, b).start()

                store_desc(p, b).start()

        for b in range(NBUF):
            store_desc(n_pos - NBUF + b, b).wait()

    return body(tokens_t, table_pad)


def kernel(tokens, table):
    if tokens.dtype != jnp.int32:
        tokens = tokens.astype(jnp.int32)
    n_vocab, emb = table.shape
    table_pad = jnp.pad(table * SCALE, ((0, 0), (0, ROW - emb)))
    out_t = _lookup(tokens.T, table_pad)
    return jnp.transpose(out_t, (2, 0, 1))


# speedup vs baseline: 1.5680x; 1.3994x over previous
import functools
import math

import jax
import jax.numpy as jnp
from jax import lax
from jax.experimental import pallas as pl
from jax.experimental.pallas import tpu as pltpu
from jax.experimental.pallas import tpu_sc as plsc

NC = 2
NS = 16
NW = NC * NS
LANES = 16
EMB = 64
SCALE = math.sqrt(EMB)
ROW = 128
NBUF = 4


@jax.jit
def _lookup(tokens_t, table_pad):
    n_pos, n_rows = tokens_t.shape
    slab = n_rows // NW

    mesh = plsc.VectorSubcoreMesh(core_axis_name="c", subcore_axis_name="s")

    pair_bufs = [pltpu.VMEM((slab, ROW), jnp.float32) for _ in range(NBUF)]
    slab_bufs = [pltpu.VMEM((EMB, slab), jnp.float32) for _ in range(NBUF)]
    gsems = [pltpu.SemaphoreType.DMA for _ in range(NBUF)]
    ssems = [pltpu.SemaphoreType.DMA for _ in range(NBUF)]

    @functools.partial(
        pl.kernel,
        out_type=jax.ShapeDtypeStruct((n_pos, EMB, n_rows), jnp.float32),
        mesh=mesh,
        scratch_types=[pltpu.VMEM((n_pos, slab), jnp.int32)]
        + pair_bufs + slab_bufs + gsems + ssems,
        compiler_params=pltpu.CompilerParams(needs_layout_passes=False),
    )
    def body(tok_hbm, table_hbm, out_hbm, tok_v, *refs):
        rows = refs[:NBUF]
        slabs = refs[NBUF:2 * NBUF]
        gsem = refs[2 * NBUF:3 * NBUF]
        ssem = refs[3 * NBUF:4 * NBUF]

        wid = lax.axis_index("s") * NC + lax.axis_index("c")
        r0 = wid * slab

        pltpu.sync_copy(tok_hbm.at[:, pl.ds(r0, slab)], tok_v)

        def gather_desc(p, b):
            return pltpu.make_async_copy(
                table_hbm.at[tok_v.at[p]], rows[b], gsem[b]
            )

        def store_desc(p, b):
            dst = out_hbm.at[p, :, pl.ds(r0, slab)]
            return pltpu.make_async_copy(slabs[b], dst, ssem[b])

        jvecs = [
            lax.iota(jnp.int32, LANES) + (g * LANES)
            for g in range(slab // LANES)
        ]

        iot = lax.iota(jnp.int32, LANES)

        def transpose_slab(b):
            @plsc.parallel_loop(0, EMB, unroll=4)
            def _e(e):
                col = jnp.bitwise_and(iot + e, EMB - 1)
                for g in range(slab // LANES):
                    v = plsc.load_gather(rows[b], [jvecs[g], col])
                    plsc.store_scatter(slabs[b], [col, jvecs[g]], v)

        for b in range(NBUF):
            gather_desc(b, b).start()

        @pl.loop(0, n_pos, step=NBUF)
        def _ring(p0):
            for b in range(NBUF):
                p = p0 + b
                gather_desc(p, b).wait()

                @pl.when(p >= NBUF)
                def _():
                    store_desc(p, b).wait()

                transpose_slab(b)

                @pl.when(p + NBUF < n_pos)
                def _():
                    gather_desc(p + NBUF, b).start()

                store_desc(p, b).start()

        for b in range(NBUF):
            store_desc(n_pos - NBUF + b, b).wait()

    return body(tokens_t, table_pad)


def kernel(tokens, table):
    if tokens.dtype != jnp.int32:
        tokens = tokens.astype(jnp.int32)
    n_vocab, emb = table.shape
    table_pad = jnp.pad(table * SCALE, ((0, 0), (0, ROW - emb)))
    out_t = _lookup(tokens.T, table_pad)
    return jnp.transpose(out_t, (2, 0, 1))


# speedup vs baseline: 3.1966x; 2.0386x over previous
import functools
import math

import jax
import jax.numpy as jnp
from jax import lax
from jax.experimental import pallas as pl
from jax.experimental.pallas import tpu as pltpu
from jax.experimental.pallas import tpu_sc as plsc

NC = 2
NS = 16
NW = NC * NS
LANES = 16
EMB = 64
SCALE = math.sqrt(EMB)
ROW = 128
NBUF = 4


@jax.jit
def _relayout(table_t, tail_rows):
    emb, n_vocab = table_t.shape
    n_tail = tail_rows.shape[0]
    n_main = n_vocab - n_tail
    n_slabs = n_main // NW // ROW * NW
    per_w = n_slabs // NW
    n_extra = n_main // ROW - n_slabs

    mesh = plsc.VectorSubcoreMesh(core_axis_name="c", subcore_axis_name="s")

    src_bufs = [pltpu.VMEM((EMB, ROW), jnp.float32) for _ in range(NBUF)]
    dst_bufs = [pltpu.VMEM((ROW, ROW), jnp.float32) for _ in range(NBUF)]
    lsems = [pltpu.SemaphoreType.DMA for _ in range(NBUF)]
    wsems = [pltpu.SemaphoreType.DMA for _ in range(NBUF)]

    @functools.partial(
        pl.kernel,
        out_type=jax.ShapeDtypeStruct((n_vocab, ROW), jnp.float32),
        mesh=mesh,
        scratch_types=src_bufs + dst_bufs + lsems + wsems,
        compiler_params=pltpu.CompilerParams(needs_layout_passes=False),
    )
    def body(tab_hbm, tail_hbm, out_hbm, *refs):
        src = refs[:NBUF]
        dst = refs[NBUF:2 * NBUF]
        lsem = refs[2 * NBUF:3 * NBUF]
        wsem = refs[3 * NBUF:4 * NBUF]

        wid = lax.axis_index("s") * NC + lax.axis_index("c")

        def col0(s):
            return pl.multiple_of(s * ROW, ROW)

        def load_desc(s, b):
            return pltpu.make_async_copy(
                tab_hbm.at[:, pl.ds(col0(s), ROW)], src[b], lsem[b]
            )

        def store_desc(s, b):
            return pltpu.make_async_copy(
                dst[b], out_hbm.at[pl.ds(col0(s), ROW)], wsem[b]
            )

        iot = lax.iota(jnp.int32, LANES)
        jvecs = [iot + (g * LANES) for g in range(ROW // LANES)]

        def transpose_slab(b):
            @plsc.parallel_loop(0, EMB, unroll=4)
            def _e(e):
                row = jnp.bitwise_and(iot + e, EMB - 1)
                for g in range(ROW // LANES):
                    v = plsc.load_gather(src[b], [row, jvecs[g]])
                    plsc.store_scatter(dst[b], [jvecs[g], row], v * SCALE)

        s0 = wid * per_w

        for b in range(NBUF):
            load_desc(s0 + b, b).start()

        @pl.loop(0, per_w, step=NBUF)
        def _ring(i0):
            for b in range(NBUF):
                i = i0 + b
                load_desc(s0 + i, b).wait()

                @pl.when(i >= NBUF)
                def _():
                    store_desc(s0 + i, b).wait()

                transpose_slab(b)

                @pl.when(i + NBUF < per_w)
                def _():
                    load_desc(s0 + i + NBUF, b).start()

                store_desc(s0 + i, b).start()

        for b in range(NBUF):
            store_desc(s0 + per_w - NBUF + b, b).wait()

        @pl.when(wid < n_extra)
        def _():
            s = n_slabs + wid
            load_desc(s, 0).start()
            load_desc(s, 0).wait()
            transpose_slab(0)
            store_desc(s, 0).start()
            store_desc(s, 0).wait()

        @pl.when(wid == n_extra)
        def _():
            pltpu.sync_copy(tail_hbm, src[0])
            pltpu.sync_copy(src[0], out_hbm.at[pl.ds(n_main, n_tail)])

    return body(table_t, tail_rows)


@jax.jit
def _lookup(tokens_t, table_pad):
    n_pos, n_rows = tokens_t.shape
    slab = n_rows // NW

    mesh = plsc.VectorSubcoreMesh(core_axis_name="c", subcore_axis_name="s")

    row_bufs = [pltpu.VMEM((slab, ROW), jnp.float32) for _ in range(NBUF)]
    slab_bufs = [pltpu.VMEM((EMB, slab), jnp.float32) for _ in range(NBUF)]
    gsems = [pltpu.SemaphoreType.DMA for _ in range(NBUF)]
    ssems = [pltpu.SemaphoreType.DMA for _ in range(NBUF)]

    @functools.partial(
        pl.kernel,
        out_type=jax.ShapeDtypeStruct((n_pos, EMB, n_rows), jnp.float32),
        mesh=mesh,
        scratch_types=[pltpu.VMEM((n_pos, slab), jnp.int32)]
        + row_bufs + slab_bufs + gsems + ssems,
        compiler_params=pltpu.CompilerParams(needs_layout_passes=False),
    )
    def body(tok_hbm, table_hbm, out_hbm, tok_v, *refs):
        rows = refs[:NBUF]
        slabs = refs[NBUF:2 * NBUF]
        gsem = refs[2 * NBUF:3 * NBUF]
        ssem = refs[3 * NBUF:4 * NBUF]

        wid = lax.axis_index("s") * NC + lax.axis_index("c")
        r0 = wid * slab

        pltpu.sync_copy(tok_hbm.at[:, pl.ds(r0, slab)], tok_v)

        def gather_desc(p, b):
            return pltpu.make_async_copy(
                table_hbm.at[tok_v.at[p]], rows[b], gsem[b]
            )

        def store_desc(p, b):
            dst = out_hbm.at[p, :, pl.ds(r0, slab)]
            return pltpu.make_async_copy(slabs[b], dst, ssem[b])

        iot = lax.iota(jnp.int32, LANES)
        jvecs = [iot + (g * LANES) for g in range(slab // LANES)]

        def transpose_slab(b):
            @plsc.parallel_loop(0, EMB, unroll=4)
            def _e(e):
                col = jnp.bitwise_and(iot + e, EMB - 1)
                for g in range(slab // LANES):
                    v = plsc.load_gather(rows[b], [jvecs[g], col])
                    plsc.store_scatter(slabs[b], [col, jvecs[g]], v)

        for b in range(NBUF):
            gather_desc(b, b).start()

        @pl.loop(0, n_pos, step=NBUF)
        def _ring(p0):
            for b in range(NBUF):
                p = p0 + b
                gather_desc(p, b).wait()

                @pl.when(p >= NBUF)
                def _():
                    store_desc(p, b).wait()

                transpose_slab(b)

                @pl.when(p + NBUF < n_pos)
                def _():
                    gather_desc(p + NBUF, b).start()

                store_desc(p, b).start()

        for b in range(NBUF):
            store_desc(n_pos - NBUF + b, b).wait()

    return body(tokens_t, table_pad)


def kernel(tokens, table):
    if tokens.dtype != jnp.int32:
        tokens = tokens.astype(jnp.int32)
    n_vocab, emb = table.shape
    n_tail = n_vocab % ROW
    tail = jnp.pad(table[n_vocab - n_tail:] * SCALE, ((0, 0), (0, ROW - emb)))
    table_pad = _relayout(table.T, tail)
    out_t = _lookup(tokens.T, table_pad)
    return jnp.transpose(out_t, (2, 0, 1))


# speedup vs baseline: 3.2592x; 1.0196x over previous
import functools
import math

import jax
import jax.numpy as jnp
from jax import lax
from jax.experimental import pallas as pl
from jax.experimental.pallas import tpu as pltpu
from jax.experimental.pallas import tpu_sc as plsc

NC = 2
NS = 16
NW = NC * NS
LANES = 16
EMB = 64
SCALE = math.sqrt(EMB)
ROW = 128
NBUF = 4


@jax.jit
def _relayout(table_t, tail_rows):
    emb, n_vocab = table_t.shape
    n_tail = tail_rows.shape[0]
    n_main = n_vocab - n_tail
    n_slabs = n_main // NW // ROW * NW
    per_w = n_slabs // NW
    n_extra = n_main // ROW - n_slabs

    mesh = plsc.VectorSubcoreMesh(core_axis_name="c", subcore_axis_name="s")

    src_bufs = [pltpu.VMEM((EMB, ROW), jnp.float32) for _ in range(NBUF)]
    dst_bufs = [pltpu.VMEM((ROW, ROW), jnp.float32) for _ in range(NBUF)]
    lsems = [pltpu.SemaphoreType.DMA for _ in range(NBUF)]
    wsems = [pltpu.SemaphoreType.DMA for _ in range(NBUF)]

    @functools.partial(
        pl.kernel,
        out_type=jax.ShapeDtypeStruct((n_vocab, ROW), jnp.float32),
        mesh=mesh,
        scratch_types=src_bufs + dst_bufs + lsems + wsems,
        compiler_params=pltpu.CompilerParams(needs_layout_passes=False),
    )
    def body(tab_hbm, tail_hbm, out_hbm, *refs):
        src = refs[:NBUF]
        dst = refs[NBUF:2 * NBUF]
        lsem = refs[2 * NBUF:3 * NBUF]
        wsem = refs[3 * NBUF:4 * NBUF]

        wid = lax.axis_index("s") * NC + lax.axis_index("c")

        def col0(s):
            return pl.multiple_of(s * ROW, ROW)

        def load_desc(s, b):
            return pltpu.make_async_copy(
                tab_hbm.at[:, pl.ds(col0(s), ROW)], src[b], lsem[b]
            )

        def store_desc(s, b):
            return pltpu.make_async_copy(
                dst[b], out_hbm.at[pl.ds(col0(s), ROW)], wsem[b]
            )

        iot = lax.iota(jnp.int32, LANES)
        jvecs = [iot + (g * LANES) for g in range(ROW // LANES)]

        def transpose_slab(b):
            @plsc.parallel_loop(0, EMB, unroll=4)
            def _e(e):
                row = jnp.bitwise_and(iot + e, EMB - 1)
                for g in range(ROW // LANES):
                    v = plsc.load_gather(src[b], [row, jvecs[g]])
                    plsc.store_scatter(dst[b], [jvecs[g], row], v)

        s0 = wid * per_w

        for b in range(NBUF):
            load_desc(s0 + b, b).start()

        @pl.loop(0, per_w, step=NBUF)
        def _ring(i0):
            for b in range(NBUF):
                i = i0 + b
                load_desc(s0 + i, b).wait()

                @pl.when(i >= NBUF)
                def _():
                    store_desc(s0 + i, b).wait()

                transpose_slab(b)

                @pl.when(i + NBUF < per_w)
                def _():
                    load_desc(s0 + i + NBUF, b).start()

                store_desc(s0 + i, b).start()

        for b in range(NBUF):
            store_desc(s0 + per_w - NBUF + b, b).wait()

        @pl.when(wid < n_extra)
        def _():
            s = n_slabs + wid
            load_desc(s, 0).start()
            load_desc(s, 0).wait()
            transpose_slab(0)
            store_desc(s, 0).start()
            store_desc(s, 0).wait()

        @pl.when(wid == n_extra)
        def _():
            pltpu.sync_copy(tail_hbm, src[0])
            pltpu.sync_copy(src[0], out_hbm.at[pl.ds(n_main, n_tail)])

    return body(table_t, tail_rows)


@jax.jit
def _lookup(tokens_t, table_pad):
    n_pos, n_rows = tokens_t.shape
    slab = n_rows // NW

    mesh = plsc.VectorSubcoreMesh(core_axis_name="c", subcore_axis_name="s")

    row_bufs = [pltpu.VMEM((slab, ROW), jnp.float32) for _ in range(NBUF)]
    slab_bufs = [pltpu.VMEM((EMB, slab), jnp.float32) for _ in range(NBUF)]
    gsems = [pltpu.SemaphoreType.DMA for _ in range(NBUF)]
    ssems = [pltpu.SemaphoreType.DMA for _ in range(NBUF)]

    @functools.partial(
        pl.kernel,
        out_type=jax.ShapeDtypeStruct((n_pos, EMB, n_rows), jnp.float32),
        mesh=mesh,
        scratch_types=[pltpu.VMEM((n_pos, slab), jnp.int32)]
        + row_bufs + slab_bufs + gsems + ssems,
        compiler_params=pltpu.CompilerParams(needs_layout_passes=False),
    )
    def body(tok_hbm, table_hbm, out_hbm, tok_v, *refs):
        rows = refs[:NBUF]
        slabs = refs[NBUF:2 * NBUF]
        gsem = refs[2 * NBUF:3 * NBUF]
        ssem = refs[3 * NBUF:4 * NBUF]

        wid = lax.axis_index("s") * NC + lax.axis_index("c")
        r0 = wid * slab

        pltpu.sync_copy(tok_hbm.at[:, pl.ds(r0, slab)], tok_v)

        def gather_desc(p, b):
            return pltpu.make_async_copy(
                table_hbm.at[tok_v.at[p]], rows[b], gsem[b]
            )

        def store_desc(p, b):
            dst = out_hbm.at[p, :, pl.ds(r0, slab)]
            return pltpu.make_async_copy(slabs[b], dst, ssem[b])

        iot = lax.iota(jnp.int32, LANES)
        jvecs = [iot + (g * LANES) for g in range(slab // LANES)]

        def transpose_slab(b):
            @plsc.parallel_loop(0, EMB, unroll=4)
            def _e(e):
                col = jnp.bitwise_and(iot + e, EMB - 1)
                for g in range(slab // LANES):
                    v = plsc.load_gather(rows[b], [jvecs[g], col])
                    plsc.store_scatter(slabs[b], [col, jvecs[g]], v * SCALE)

        for b in range(NBUF):
            gather_desc(b, b).start()

        @pl.loop(0, n_pos, step=NBUF)
        def _ring(p0):
            for b in range(NBUF):
                p = p0 + b
                gather_desc(p, b).wait()

                @pl.when(p >= NBUF)
                def _():
                    store_desc(p, b).wait()

                transpose_slab(b)

                @pl.when(p + NBUF < n_pos)
                def _():
                    gather_desc(p + NBUF, b).start()

                store_desc(p, b).start()

        for b in range(NBUF):
            store_desc(n_pos - NBUF + b, b).wait()

    return body(tokens_t, table_pad)


def kernel(tokens, table):
    if tokens.dtype != jnp.int32:
        tokens = tokens.astype(jnp.int32)
    n_vocab, emb = table.shape
    n_tail = n_vocab % ROW
    tail = jnp.pad(table[n_vocab - n_tail:], ((0, 0), (0, ROW - emb)))
    table_pad = _relayout(table.T, tail)
    out_t = _lookup(tokens.T, table_pad)
    return jnp.transpose(out_t, (2, 0, 1))
